# Initial kernel scaffold; baseline (speedup 1.0000x reference)
import jax
import jax.numpy as jnp
from jax.experimental import pallas as pl


def kernel(x, pooled):
    raise NotImplementedError("write your pallas kernel here")


# speedup vs baseline: 46.1721x; 46.1721x over previous
import jax
import jax.numpy as jnp
from jax.experimental import pallas as pl


def _body(x_ref, p_ref, o_ref):
    R = p_ref.shape[1]
    xv = x_ref[0]
    p = p_ref[0]
    xw = xv.reshape(R, 2, 112, 2, 96)
    v00 = xw[:, 0, :, 0, :]
    v01 = xw[:, 0, :, 1, :]
    v10 = xw[:, 1, :, 0, :]
    v11 = xw[:, 1, :, 1, :]
    m = jnp.maximum(jnp.maximum(v00, v01), jnp.maximum(v10, v11))
    r = jnp.maximum(p, 0.0)
    z = jnp.zeros_like(r)
    n00 = v00 == m
    n01 = v01 == m
    n10 = v10 == m
    o00 = jnp.where(n00, r, z)
    o01 = jnp.where(n01 & ~n00, r, z)
    o10 = jnp.where(n10 & ~n00 & ~n01, r, z)
    o11 = jnp.where(~(n00 | n01 | n10), r, z)
    row0 = jnp.stack([o00, o01], axis=2).reshape(R, 224, 96)
    row1 = jnp.stack([o10, o11], axis=2).reshape(R, 224, 96)
    o_ref[0] = jnp.stack([row0, row1], axis=1).reshape(2 * R, 224, 96)


def kernel(x, pooled):
    B, H, W, C = x.shape
    Hp, Wp = H // 2, W // 2
    R = 8
    grid = (B, Hp // R)
    return pl.pallas_call(
        _body,
        grid=grid,
        in_specs=[
            pl.BlockSpec((1, 2 * R, W, C), lambda b, i: (b, i, 0, 0)),
            pl.BlockSpec((1, R, Wp, C), lambda b, i: (b, i, 0, 0)),
        ],
        out_specs=pl.BlockSpec((1, 2 * R, W, C), lambda b, i: (b, i, 0, 0)),
        out_shape=jax.ShapeDtypeStruct((B, H, W, C), x.dtype),
    )(x, pooled)


# speedup vs baseline: 79.4984x; 1.7218x over previous
import jax
import jax.numpy as jnp
from jax.experimental import pallas as pl


def _body(x_ref, p_ref, o_ref):
    R = p_ref.shape[1]
    W = x_ref.shape[2]
    xv = x_ref[0]
    p = p_ref[0]
    even = jax.lax.broadcasted_iota(jnp.int32, xv.shape, 1) % 2 == 0
    partner = jnp.where(even, jnp.roll(xv, -1, axis=1), jnp.roll(xv, 1, axis=1))
    mw = jnp.maximum(xv, partner)
    mwr = mw.reshape(R, 2, *xv.shape[1:])
    m = jnp.maximum(mwr[:, 0], mwr[:, 1])
    xr = xv.reshape(R, 2, *xv.shape[1:])
    qr = partner.reshape(R, 2, *xv.shape[1:])
    x0 = xr[:, 0]
    x1 = xr[:, 1]
    q0 = qr[:, 0]
    q1 = qr[:, 1]
    eqa0 = x0 >= m
    eqw0 = q0 >= m
    eqa1 = x1 >= m
    eqw1 = q1 >= m
    odd = jax.lax.broadcasted_iota(jnp.int32, x0.shape, 1) % 2 == 1
    win0 = eqa0 & ~(odd & eqw0)
    win1 = eqa1 & ~(odd & eqw1) & ~(eqa0 | eqw0)
    rp = jnp.maximum(p, 0.0)
    rup = jnp.stack([rp, rp], axis=2).reshape(R, W, p.shape[2])
    z = jnp.zeros_like(rup)
    o0 = jnp.where(win0, rup, z)
    o1 = jnp.where(win1, rup, z)
    o_ref[0] = jnp.stack([o0, o1], axis=1).reshape(xv.shape)


def kernel(x, pooled):
    B, H, W, C = x.shape
    Hp, Wp = H // 2, W // 2
    R = 8
    grid = (B, Hp // R)
    return pl.pallas_call(
        _body,
        grid=grid,
        in_specs=[
            pl.BlockSpec((1, 2 * R, W, C), lambda b, i: (b, i, 0, 0)),
            pl.BlockSpec((1, R, Wp, C), lambda b, i: (b, i, 0, 0)),
        ],
        out_specs=pl.BlockSpec((1, 2 * R, W, C), lambda b, i: (b, i, 0, 0)),
        out_shape=jax.ShapeDtypeStruct((B, H, W, C), x.dtype),
    )(x, pooled)


# speedup vs baseline: 83.2864x; 1.0476x over previous
import jax
import jax.numpy as jnp
from jax.experimental import pallas as pl


def _body(x_ref, p_ref, o_ref):
    R = p_ref.shape[1]
    W = x_ref.shape[2]
    xv = x_ref[0]
    p = p_ref[0]
    even = jax.lax.broadcasted_iota(jnp.int32, xv.shape, 1) % 2 == 0
    partner = jnp.where(even, jnp.roll(xv, -1, axis=1), jnp.roll(xv, 1, axis=1))
    mw = jnp.maximum(xv, partner)
    mwr = mw.reshape(R, 2, *xv.shape[1:])
    m = jnp.maximum(mwr[:, 0], mwr[:, 1])
    xr = xv.reshape(R, 2, *xv.shape[1:])
    qr = partner.reshape(R, 2, *xv.shape[1:])
    x0 = xr[:, 0]
    x1 = xr[:, 1]
    q0 = qr[:, 0]
    q1 = qr[:, 1]
    eqa0 = x0 >= m
    eqw0 = q0 >= m
    eqa1 = x1 >= m
    eqw1 = q1 >= m
    odd = jax.lax.broadcasted_iota(jnp.int32, x0.shape, 1) % 2 == 1
    win0 = eqa0 & ~(odd & eqw0)
    win1 = eqa1 & ~(odd & eqw1) & ~(eqa0 | eqw0)
    rp = jnp.maximum(p, 0.0)
    rup = jnp.stack([rp, rp], axis=2).reshape(R, W, p.shape[2])
    z = jnp.zeros_like(rup)
    o0 = jnp.where(win0, rup, z)
    o1 = jnp.where(win1, rup, z)
    o_ref[0] = jnp.stack([o0, o1], axis=1).reshape(xv.shape)


def kernel(x, pooled):
    B, H, W, C = x.shape
    Hp, Wp = H // 2, W // 2
    R = 28
    grid = (B, Hp // R)
    return pl.pallas_call(
        _body,
        grid=grid,
        in_specs=[
            pl.BlockSpec((1, 2 * R, W, C), lambda b, i: (b, i, 0, 0)),
            pl.BlockSpec((1, R, Wp, C), lambda b, i: (b, i, 0, 0)),
        ],
        out_specs=pl.BlockSpec((1, 2 * R, W, C), lambda b, i: (b, i, 0, 0)),
        out_shape=jax.ShapeDtypeStruct((B, H, W, C), x.dtype),
    )(x, pooled)
